# baseline (speedup 1.0000x reference)
import functools

import jax
import jax.numpy as jnp
from jax import lax
from jax.experimental import pallas as pl
from jax.experimental.pallas import tpu as pltpu
from jax.experimental.pallas import tpu_sc as plsc

NC = 2
NS = 16
NW = NC * NS
D = 64
C = 512
G = 128


def _run(n_w, xf, emb, bias_f, *, interpret=False):
    n = xf.shape[0]
    n_chunks = n_w // C
    mesh = plsc.VectorSubcoreMesh(
        core_axis_name="c", subcore_axis_name="s", num_cores=NC, num_subcores=NS
    )

    @functools.partial(
        pl.kernel,
        out_type=jax.ShapeDtypeStruct((n, D), jnp.float32),
        mesh=mesh,
        scratch_types=[
            pltpu.VMEM((C,), jnp.int32),
            pltpu.VMEM((C,), jnp.float32),
            pltpu.VMEM((C, D), jnp.float32),
            pltpu.SemaphoreType.DMA,
        ],
        compiler_params=pltpu.CompilerParams(use_tc_tiling_on_sc=False),
        interpret=interpret,
    )
    def run(x_hbm, emb_hbm, bias_hbm, out_hbm, idx_v, bias_v, rows_v, sem):
        wid = lax.axis_index("s") * NC + lax.axis_index("c")
        base = wid * n_w

        def chunk_body(c, _):
            off = base + c * C
            pltpu.sync_copy(x_hbm.at[pl.ds(off, C)], idx_v)
            copies = []
            for g in range(C // G):
                sl = pl.ds(g * G, G)
                copies.append(
                    pltpu.async_copy(emb_hbm.at[idx_v.at[sl]], rows_v.at[sl], sem)
                )
                copies.append(
                    pltpu.async_copy(bias_hbm.at[idx_v.at[sl]], bias_v.at[sl], sem)
                )
            for cp in copies:
                cp.wait()

            def row_body(r16, _):
                bv16 = bias_v[pl.ds(r16 * 16, 16)]
                for j in range(16):
                    r = r16 * 16 + j
                    bv = jnp.broadcast_to(bv16[j], (16,))
                    for q in range(4):
                        col = pl.ds(q * 16, 16)
                        rows_v[r, col] = rows_v[r, col] + bv
                return 0

            lax.fori_loop(0, C // 16, row_body, 0, unroll=False)
            pltpu.sync_copy(rows_v, out_hbm.at[pl.ds(off, C)])
            return 0

        lax.fori_loop(0, n_chunks, chunk_body, 0, unroll=False)

    return run(xf, emb, bias_f)


def kernel(x, embedding_weight, bias_weight):
    b, f = x.shape
    n = b * f
    n_w = n // NW
    assert n % NW == 0 and n_w % C == 0
    xf = x.reshape(n)
    bias_f = bias_weight.reshape(-1)
    out = _run(n_w, xf, embedding_weight, bias_f)
    return out.reshape(b, f, D)
